# baseline (speedup 1.0000x reference)
import jax
import jax.numpy as jnp
from jax import lax
from jax.experimental import pallas as pl
from jax.experimental.pallas import tpu as pltpu
from jax.experimental.pallas import tpu_sc as plsc

NUM_EMB = 1000000
DIM = 64
BATCH = 16384

_NC = 2
_NS = 16
_NW = _NC * _NS
_B_PER_W = BATCH // _NW
_O_PER_W = _B_PER_W * DIM // 2 // 128


def _body(w2_hbm, idx_hbm, out_hbm, idx_v, pidx_v, rows_v, obuf, sem):
    wid = lax.axis_index("s") * _NC + lax.axis_index("c")
    base = wid * _B_PER_W

    pltpu.sync_copy(idx_hbm.at[pl.ds(base, _B_PER_W)], idx_v)

    def make_pair_idx(j, _):
        v = idx_v[pl.ds(j * 16, 16)]
        pidx_v[pl.ds(j * 16, 16)] = lax.shift_right_logical(v, 1)
        return 0

    lax.fori_loop(0, _B_PER_W // 16, make_pair_idx, 0)

    pltpu.async_copy(w2_hbm.at[pidx_v], rows_v, sem).wait()

    iota = lax.iota(jnp.int32, 16)

    def cast_half_row(i, _):
        r = i // 2
        rr = jnp.full((16,), r, jnp.int32)
        par = lax.bitwise_and(plsc.load_gather(idx_v, [rr]), 1) * DIM
        col = par + (i % 2) * 32 + 2 * iota
        ev = plsc.load_gather(rows_v, [rr, col])
        od = plsc.load_gather(rows_v, [rr, col + 1])
        packed = plsc.pack(ev, od, format=plsc.PackFormat.INTERLEAVED)
        words = plsc.bitcast(packed, jnp.int32)
        obuf[i // 8, pl.ds((i % 8) * 16, 16)] = words
        return 0

    lax.fori_loop(0, _B_PER_W * 2, cast_half_row, 0)

    pltpu.sync_copy(obuf, out_hbm.at[pl.ds(wid * _O_PER_W, _O_PER_W)])


_sc_gather_cast = pl.kernel(
    _body,
    out_type=jax.ShapeDtypeStruct((BATCH * DIM // 2 // 128, 128), jnp.int32),
    mesh=plsc.VectorSubcoreMesh(
        core_axis_name="c", subcore_axis_name="s",
        num_cores=_NC, num_subcores=_NS),
    compiler_params=pltpu.CompilerParams(needs_layout_passes=False),
    scratch_types=[
        pltpu.VMEM((_B_PER_W,), jnp.int32),
        pltpu.VMEM((_B_PER_W,), jnp.int32),
        pltpu.VMEM((_B_PER_W, 2 * DIM), jnp.float32),
        pltpu.VMEM((_O_PER_W, 128), jnp.int32),
        pltpu.SemaphoreType.DMA,
    ],
)


def kernel(weights, inputs, train):
    del train
    w2 = weights.reshape(NUM_EMB // 2, 2 * DIM)
    raw = _sc_gather_cast(w2, inputs)
    out = jax.lax.bitcast_convert_type(raw, jnp.bfloat16)
    return out.reshape(BATCH, DIM)


# speedup vs baseline: 1.3857x; 1.3857x over previous
import jax
import jax.numpy as jnp
from jax import lax
from jax.experimental import pallas as pl
from jax.experimental.pallas import tpu as pltpu
from jax.experimental.pallas import tpu_sc as plsc

NUM_EMB = 1000000
DIM = 64
BATCH = 16384

_NC = 2
_NS = 16
_NW = _NC * _NS
_B_PER_W = BATCH // _NW
_O_PER_W = _B_PER_W * DIM // 2 // 128


def _body(w_hbm, idx_hbm, out_hbm, idx_v, rows_v, obuf, sem):
    wid = lax.axis_index("s") * _NC + lax.axis_index("c")
    base = wid * _B_PER_W

    pltpu.sync_copy(idx_hbm.at[pl.ds(base, _B_PER_W)], idx_v)

    def fire(j, _):
        v = idx_v[pl.ds(j * 16, 16)]
        for k in range(16):
            pltpu.async_copy(
                w_hbm.at[pl.ds(v[k], 1)],
                rows_v.at[pl.ds(j * 16 + k, 1)],
                sem)
        return 0

    lax.fori_loop(0, _B_PER_W // 16, fire, 0)
    pltpu.make_async_copy(
        w_hbm.at[pl.ds(0, _B_PER_W)], rows_v, sem).wait()

    iota = lax.iota(jnp.int32, 16)

    def cast_half_row(i, _):
        r = i // 2
        rr = jnp.full((16,), r, jnp.int32)
        col = (i % 2) * 32 + 2 * iota
        ev = plsc.load_gather(rows_v, [rr, col])
        od = plsc.load_gather(rows_v, [rr, col + 1])
        packed = plsc.pack(ev, od, format=plsc.PackFormat.INTERLEAVED)
        words = plsc.bitcast(packed, jnp.int32)
        obuf[i // 8, pl.ds((i % 8) * 16, 16)] = words
        return 0

    lax.fori_loop(0, _B_PER_W * 2, cast_half_row, 0)

    pltpu.sync_copy(obuf, out_hbm.at[pl.ds(wid * _O_PER_W, _O_PER_W)])


_sc_gather_cast = pl.kernel(
    _body,
    out_type=jax.ShapeDtypeStruct((BATCH * DIM // 2 // 128, 128), jnp.int32),
    mesh=plsc.VectorSubcoreMesh(
        core_axis_name="c", subcore_axis_name="s",
        num_cores=_NC, num_subcores=_NS),
    compiler_params=pltpu.CompilerParams(needs_layout_passes=False),
    scratch_types=[
        pltpu.VMEM((_B_PER_W,), jnp.int32),
        pltpu.VMEM((_B_PER_W, DIM), jnp.float32),
        pltpu.VMEM((_O_PER_W, 128), jnp.int32),
        pltpu.SemaphoreType.DMA,
    ],
)


def kernel(weights, inputs, train):
    del train
    raw = _sc_gather_cast(weights, inputs)
    out = jax.lax.bitcast_convert_type(raw, jnp.bfloat16)
    return out.reshape(BATCH, DIM)


# speedup vs baseline: 1.4787x; 1.0671x over previous
import jax
import jax.numpy as jnp
from jax import lax
from jax.experimental import pallas as pl
from jax.experimental.pallas import tpu as pltpu
from jax.experimental.pallas import tpu_sc as plsc

NUM_EMB = 1000000
DIM = 64
BATCH = 16384

_NC = 2
_NS = 16
_NW = _NC * _NS
_B_PER_W = BATCH // _NW
_NPAIR = DIM // 2


def _body(w_hbm, idx_hbm, out_hbm, idx_v, rows_v, obuf, sem):
    wid = lax.axis_index("s") * _NC + lax.axis_index("c")
    base = wid * _B_PER_W

    pltpu.sync_copy(idx_hbm.at[pl.ds(base, _B_PER_W)], idx_v)
    pltpu.async_copy(w_hbm.at[idx_v], rows_v, sem).wait()

    iota = lax.iota(jnp.int32, 16)

    def cast_group(i, _):
        k = i // (_B_PER_W // 16)
        g = i % (_B_PER_W // 16)
        pos = g * 16 + iota
        ev = plsc.load_gather(rows_v, [pos, jnp.full((16,), 2 * k, jnp.int32)])
        od = plsc.load_gather(rows_v, [pos, jnp.full((16,), 2 * k + 1, jnp.int32)])
        packed = plsc.pack(ev, od, format=plsc.PackFormat.INTERLEAVED)
        obuf[k, pl.ds(g * 16, 16)] = plsc.bitcast(packed, jnp.int32)
        return 0

    lax.fori_loop(0, _NPAIR * (_B_PER_W // 16), cast_group, 0)

    pltpu.sync_copy(obuf, out_hbm.at[:, pl.ds(base, _B_PER_W)])


_sc_gather_cast = pl.kernel(
    _body,
    out_type=jax.ShapeDtypeStruct((_NPAIR, BATCH), jnp.int32),
    mesh=plsc.VectorSubcoreMesh(
        core_axis_name="c", subcore_axis_name="s",
        num_cores=_NC, num_subcores=_NS),
    compiler_params=pltpu.CompilerParams(
        needs_layout_passes=False, use_tc_tiling_on_sc=False),
    scratch_types=[
        pltpu.VMEM((_B_PER_W,), jnp.int32),
        pltpu.VMEM((_B_PER_W, DIM), jnp.float32),
        pltpu.VMEM((_NPAIR, _B_PER_W), jnp.int32),
        pltpu.SemaphoreType.DMA,
    ],
)


def kernel(weights, inputs, train):
    del train
    raw = _sc_gather_cast(weights, inputs)
    pairs = jax.lax.bitcast_convert_type(raw, jnp.bfloat16)
    return pairs.transpose(1, 0, 2).reshape(BATCH, DIM)


# speedup vs baseline: 3.7199x; 2.5156x over previous
import jax
import jax.numpy as jnp
from jax import lax
from jax.experimental import pallas as pl
from jax.experimental.pallas import tpu as pltpu
from jax.experimental.pallas import tpu_sc as plsc

NUM_EMB = 1000000
DIM = 64
BATCH = 16384

_NC = 2
_NS = 16
_NW = _NC * _NS
_B_PER_W = BATCH // _NW
_NPAIR = DIM // 2
_CH = 4
_NCHUNK = _B_PER_W // _CH


def _body(wt_hbm, idx_hbm, out_hbm, idx_v, b0, b1, obuf, s0, s1):
    wid = lax.axis_index("s") * _NC + lax.axis_index("c")
    base = wid * _B_PER_W

    pltpu.sync_copy(idx_hbm.at[pl.ds(base, _B_PER_W)],
                    idx_v.at[pl.ds(0, _B_PER_W)])

    iota = lax.iota(jnp.int32, 16)
    mask4 = iota < _CH
    lsel = lax.bitwise_and(iota, _CH - 1)

    def fire(c, buf, sem):
        v = idx_v[pl.ds(c * _CH, 16)]
        for k in range(_CH):
            col = lax.shift_right_logical(v[k], 7) * 128
            pltpu.async_copy(
                wt_hbm.at[:, pl.ds(col, 128)], buf.at[k], sem)

    def drain(buf, sem):
        for k in range(_CH):
            pltpu.make_async_copy(
                wt_hbm.at[:, pl.ds(0, 128)], buf.at[k], sem).wait()

    def extract(c, buf):
        v = idx_v[pl.ds(c * _CH, 16)]
        lanevec = lax.bitwise_and(v, 127)

        def word_row(k2, _):
            ev = plsc.load_gather(
                buf, [lsel, jnp.full((16,), 2 * k2, jnp.int32), lanevec],
                mask=mask4)
            od = plsc.load_gather(
                buf, [lsel, jnp.full((16,), 2 * k2 + 1, jnp.int32), lanevec],
                mask=mask4)
            packed = plsc.pack(ev, od, format=plsc.PackFormat.INTERLEAVED)
            words = plsc.bitcast(packed, jnp.int32)
            plsc.store_scatter(
                obuf, [jnp.full((16,), k2, jnp.int32), c * _CH + iota],
                words, mask=mask4)
            return 0

        lax.fori_loop(0, _NPAIR, word_row, 0)

    fire(0, b0, s0)
    fire(1, b1, s1)

    def step(i, _):
        c0 = 2 * i
        drain(b0, s0)
        extract(c0, b0)

        @pl.when(c0 + 2 < _NCHUNK)
        def _():
            fire(c0 + 2, b0, s0)

        drain(b1, s1)
        extract(c0 + 1, b1)

        @pl.when(c0 + 3 < _NCHUNK)
        def _():
            fire(c0 + 3, b1, s1)

        return 0

    lax.fori_loop(0, _NCHUNK // 2, step, 0)

    pltpu.sync_copy(obuf, out_hbm.at[:, pl.ds(base, _B_PER_W)])


_sc_gather_cast = pl.kernel(
    _body,
    out_type=jax.ShapeDtypeStruct((_NPAIR, BATCH), jnp.int32),
    mesh=plsc.VectorSubcoreMesh(
        core_axis_name="c", subcore_axis_name="s",
        num_cores=_NC, num_subcores=_NS),
    compiler_params=pltpu.CompilerParams(needs_layout_passes=False),
    scratch_types=[
        pltpu.VMEM((_B_PER_W + 16,), jnp.int32),
        pltpu.VMEM((_CH, DIM, 128), jnp.float32),
        pltpu.VMEM((_CH, DIM, 128), jnp.float32),
        pltpu.VMEM((_NPAIR, _B_PER_W), jnp.int32),
        pltpu.SemaphoreType.DMA,
        pltpu.SemaphoreType.DMA,
    ],
)


def kernel(weights, inputs, train):
    del train
    wt = weights.T
    raw = _sc_gather_cast(wt, inputs)
    pairs = jax.lax.bitcast_convert_type(raw, jnp.bfloat16)
    return pairs.transpose(1, 0, 2).reshape(BATCH, DIM)


# speedup vs baseline: 4.0651x; 1.0928x over previous
import jax
import jax.numpy as jnp
from jax import lax
from jax.experimental import pallas as pl
from jax.experimental.pallas import tpu as pltpu
from jax.experimental.pallas import tpu_sc as plsc

NUM_EMB = 1000000
DIM = 64
BATCH = 16384

_NC = 2
_NS = 16
_NW = _NC * _NS
_B_PER_W = BATCH // _NW
_NPAIR = DIM // 2
_CH = 4
_NCHUNK = _B_PER_W // _CH


def _body(wt_hbm, idx_hbm, out_hbm, idx_v, b0, b1, b2, obuf, s0, s1, s2):
    wid = lax.axis_index("s") * _NC + lax.axis_index("c")
    base = wid * _B_PER_W

    pltpu.sync_copy(idx_hbm.at[pl.ds(base, _B_PER_W)],
                    idx_v.at[pl.ds(0, _B_PER_W)])

    iota = lax.iota(jnp.int32, 16)
    mask4 = iota < _CH
    lsel = lax.bitwise_and(iota, _CH - 1)

    def fire(c, buf, sem):
        v = idx_v[pl.ds(c * _CH, 16)]
        for k in range(_CH):
            col = lax.shift_right_logical(v[k], 7) * 128
            pltpu.async_copy(
                wt_hbm.at[:, pl.ds(col, 128)], buf.at[k], sem)

    def drain(buf, sem):
        for k in range(_CH):
            pltpu.make_async_copy(
                wt_hbm.at[:, pl.ds(0, 128)], buf.at[k], sem).wait()

    def extract(c, buf):
        v = idx_v[pl.ds(c * _CH, 16)]
        lanevec = lax.bitwise_and(v, 127)

        def word_row(k2, _):
            ev = plsc.load_gather(
                buf, [lsel, jnp.full((16,), 2 * k2, jnp.int32), lanevec],
                mask=mask4)
            od = plsc.load_gather(
                buf, [lsel, jnp.full((16,), 2 * k2 + 1, jnp.int32), lanevec],
                mask=mask4)
            packed = plsc.pack(ev, od, format=plsc.PackFormat.INTERLEAVED)
            words = plsc.bitcast(packed, jnp.int32)
            plsc.store_scatter(
                obuf, [jnp.full((16,), k2, jnp.int32), c * _CH + iota],
                words, mask=mask4)
            return 0

        lax.fori_loop(0, _NPAIR, word_row, 0)

    fire(0, b0, s0)
    fire(1, b1, s1)
    fire(2, b2, s2)

    def step(i, _):
        for j, (b, s) in enumerate(((b0, s0), (b1, s1), (b2, s2))):
            c = 3 * i + j
            drain(b, s)
            extract(c, b)

            @pl.when(c + 3 < _NCHUNK)
            def _():
                fire(c + 3, b, s)

        return 0

    lax.fori_loop(0, _NCHUNK // 3, step, 0)
    drain(b0, s0)
    extract(_NCHUNK - 2, b0)
    drain(b1, s1)
    extract(_NCHUNK - 1, b1)

    pltpu.sync_copy(obuf, out_hbm.at[:, pl.ds(base, _B_PER_W)])


_sc_gather_cast = pl.kernel(
    _body,
    out_type=jax.ShapeDtypeStruct((_NPAIR, BATCH), jnp.int32),
    mesh=plsc.VectorSubcoreMesh(
        core_axis_name="c", subcore_axis_name="s",
        num_cores=_NC, num_subcores=_NS),
    compiler_params=pltpu.CompilerParams(needs_layout_passes=False),
    scratch_types=[
        pltpu.VMEM((_B_PER_W + 16,), jnp.int32),
        pltpu.VMEM((_CH, DIM, 128), jnp.float32),
        pltpu.VMEM((_CH, DIM, 128), jnp.float32),
        pltpu.VMEM((_CH, DIM, 128), jnp.float32),
        pltpu.VMEM((_NPAIR, _B_PER_W), jnp.int32),
        pltpu.SemaphoreType.DMA,
        pltpu.SemaphoreType.DMA,
        pltpu.SemaphoreType.DMA,
    ],
)


def kernel(weights, inputs, train):
    del train
    wt = weights.T
    raw = _sc_gather_cast(wt, inputs)
    pairs = jax.lax.bitcast_convert_type(raw, jnp.bfloat16)
    return pairs.transpose(1, 0, 2).reshape(BATCH, DIM)
